# baseline (speedup 1.0000x reference)
import functools

import jax
import jax.numpy as jnp
from jax import lax
from jax.experimental import pallas as pl
from jax.experimental.pallas import tpu as pltpu
from jax.experimental.pallas import tpu_sc as plsc

ROWS_PER_CHUNK = 4
NW = 32
LANES = 128
SUB = 8


@functools.lru_cache(maxsize=None)
def _build(num_rows: int, seq: int):
    seq_pad = (seq + SUB - 1) // SUB * SUB
    rows_per_w = num_rows // NW
    chunks = rows_per_w // ROWS_PER_CHUNK

    mesh = plsc.VectorSubcoreMesh(core_axis_name="c", subcore_axis_name="s")

    @functools.partial(
        pl.kernel,
        mesh=mesh,
        out_type=jax.ShapeDtypeStruct((num_rows, seq_pad, LANES), jnp.float32),
        scratch_types=[
            pltpu.VMEM((rows_per_w, seq), jnp.int32),
            pltpu.VMEM((ROWS_PER_CHUNK, seq_pad, LANES), jnp.float32),
            pltpu.VMEM((ROWS_PER_CHUNK, seq_pad, LANES), jnp.float32),
            pltpu.SemaphoreType.DMA,
            pltpu.SemaphoreType.DMA,
            pltpu.SemaphoreType.DMA,
            pltpu.SemaphoreType.DMA,
        ],
        compiler_params=pltpu.CompilerParams(use_tc_tiling_on_sc=True),
    )
    def gather_kernel(idx_hbm, table_hbm, out_hbm, idx_v, rows0, rows1,
                      gsem0, gsem1, ssem0, ssem1):
        wid = lax.axis_index("s") * 2 + lax.axis_index("c")
        row_base = wid * rows_per_w
        bufs = (rows0, rows1)
        gsems = (gsem0, gsem1)
        ssems = (ssem0, ssem1)

        pltpu.sync_copy(idx_hbm.at[pl.ds(row_base, rows_per_w)], idx_v)

        def fire_gathers(g, b):
            for j in range(ROWS_PER_CHUNK):
                pltpu.async_copy(
                    table_hbm.at[idx_v.at[g * ROWS_PER_CHUNK + j]],
                    bufs[b].at[j, pl.ds(0, seq)],
                    gsems[b],
                )

        def drain_gathers(b):
            for j in range(ROWS_PER_CHUNK):
                pltpu.make_async_copy(
                    table_hbm.at[idx_v.at[0]],
                    bufs[b].at[j, pl.ds(0, seq)],
                    gsems[b],
                ).wait()

        def store_chunk(g, b):
            pltpu.async_copy(
                bufs[b],
                out_hbm.at[pl.ds(row_base + g * ROWS_PER_CHUNK, ROWS_PER_CHUNK)],
                ssems[b],
            )

        def wait_store(b):
            pltpu.make_async_copy(
                bufs[b],
                out_hbm.at[pl.ds(row_base, ROWS_PER_CHUNK)],
                ssems[b],
            ).wait()

        fire_gathers(0, 0)

        def body(i, carry):
            for b in range(2):
                g = 2 * i + b
                nb = 1 - b
                @pl.when(g >= 1)
                def _():
                    wait_store(nb)

                @pl.when(g + 1 < chunks)
                def _():
                    fire_gathers(g + 1, nb)

                drain_gathers(b)
                store_chunk(g, b)
            return carry

        lax.fori_loop(0, chunks // 2, body, 0)

        wait_store((chunks - 1) % 2)

    return gather_kernel


def kernel(input_, weight):
    b, s = input_.shape
    d = weight.shape[1]
    wpad = jnp.pad(weight, ((0, 0), (0, LANES - d)))
    idx = input_.astype(jnp.int32)
    h = b // 2
    run = _build(h, s)
    o1 = run(idx[:h], wpad)
    o2 = run(idx[h:], wpad)
    return jnp.concatenate([o1[:, :s, :d], o2[:, :s, :d]], axis=0)


# speedup vs baseline: 1.1356x; 1.1356x over previous
import functools

import jax
import jax.numpy as jnp
from jax import lax
from jax.experimental import pallas as pl
from jax.experimental.pallas import tpu as pltpu
from jax.experimental.pallas import tpu_sc as plsc

ROWS_PER_CHUNK = 4
NW = 32
LANES = 128
SUB = 8


@functools.lru_cache(maxsize=None)
def _build(num_rows: int, seq: int):
    seq_pad = (seq + SUB - 1) // SUB * SUB
    rows_per_w = num_rows // NW
    chunks = rows_per_w // ROWS_PER_CHUNK

    mesh = plsc.VectorSubcoreMesh(core_axis_name="c", subcore_axis_name="s")

    @functools.partial(
        pl.kernel,
        mesh=mesh,
        out_type=jax.ShapeDtypeStruct((num_rows, seq_pad, LANES), jnp.float32),
        scratch_types=[
            pltpu.VMEM((rows_per_w, seq), jnp.int32),
            pltpu.VMEM((ROWS_PER_CHUNK, seq_pad, LANES), jnp.float32),
            pltpu.VMEM((ROWS_PER_CHUNK, seq_pad, LANES), jnp.float32),
            pltpu.SemaphoreType.DMA,
            pltpu.SemaphoreType.DMA,
            pltpu.SemaphoreType.DMA,
            pltpu.SemaphoreType.DMA,
        ],
        compiler_params=pltpu.CompilerParams(use_tc_tiling_on_sc=True),
    )
    def gather_kernel(idx_hbm, table_hbm, out_hbm, idx_v, rows0, rows1,
                      gsem0, gsem1, ssem0, ssem1):
        wid = lax.axis_index("s") * 2 + lax.axis_index("c")
        row_base = wid * rows_per_w
        bufs = (rows0, rows1)
        gsems = (gsem0, gsem1)
        ssems = (ssem0, ssem1)

        pltpu.sync_copy(idx_hbm.at[pl.ds(row_base, rows_per_w)], idx_v)

        def fire_gathers(g, b):
            for j in range(ROWS_PER_CHUNK):
                pltpu.async_copy(
                    table_hbm.at[idx_v.at[g * ROWS_PER_CHUNK + j]],
                    bufs[b].at[j, pl.ds(0, seq)],
                    gsems[b],
                )

        def drain_gathers(b):
            for j in range(ROWS_PER_CHUNK):
                pltpu.make_async_copy(
                    table_hbm.at[idx_v.at[0]],
                    bufs[b].at[j, pl.ds(0, seq)],
                    gsems[b],
                ).wait()

        def store_chunk(g, b):
            pltpu.async_copy(
                bufs[b],
                out_hbm.at[pl.ds(row_base + g * ROWS_PER_CHUNK, ROWS_PER_CHUNK)],
                ssems[b],
            )

        def wait_store(b):
            pltpu.make_async_copy(
                bufs[b],
                out_hbm.at[pl.ds(row_base, ROWS_PER_CHUNK)],
                ssems[b],
            ).wait()

        fire_gathers(0, 0)

        def body(i, carry):
            for b in range(2):
                g = 2 * i + b
                nb = 1 - b
                @pl.when(g >= 1)
                def _():
                    wait_store(nb)

                @pl.when(g + 1 < chunks)
                def _():
                    fire_gathers(g + 1, nb)

                drain_gathers(b)
                store_chunk(g, b)
            return carry

        lax.fori_loop(0, chunks // 2, body, 0)

        wait_store((chunks - 1) % 2)

    return gather_kernel


def kernel(input_, weight):
    b, s = input_.shape
    d = weight.shape[1]
    wpad = jnp.pad(weight, ((0, 0), (0, LANES - d)))
    outp = _build(b, s)(input_.astype(jnp.int32), wpad)
    return outp[:, :s, :d]
